# baseline (speedup 1.0000x reference)
import functools

import jax
import jax.numpy as jnp
from jax import lax
from jax.experimental import pallas as pl
from jax.experimental.pallas import tpu as pltpu
from jax.experimental.pallas import tpu_sc as plsc

_LANES = 16


def _make_sc_add(batch, maxlen, dim):
    info = plsc.get_sparse_core_info()
    nc, ns = info.num_cores, info.num_subcores
    nw = nc * ns
    assert batch % nw == 0 and dim % _LANES == 0
    b_per_w = batch // nw
    d_chunks = dim // _LANES

    mesh = plsc.VectorSubcoreMesh(core_axis_name="c", subcore_axis_name="s")

    @functools.partial(
        pl.kernel,
        out_type=jax.ShapeDtypeStruct((batch, maxlen, dim), jnp.float32),
        mesh=mesh,
        scratch_types=[
            pltpu.VMEM((maxlen, dim), jnp.float32),
            pltpu.VMEM((maxlen, dim), jnp.float32),
            pltpu.VMEM((maxlen, dim), jnp.float32),
            pltpu.VMEM((maxlen, dim), jnp.float32),
            pltpu.VMEM((maxlen, dim), jnp.float32),
            pltpu.SemaphoreType.DMA,
            pltpu.SemaphoreType.DMA,
            pltpu.SemaphoreType.DMA,
            pltpu.SemaphoreType.DMA,
        ],
    )
    def sc_add(x_hbm, pos_hbm, out_hbm, pos_v, ib0, ib1, ob0, ob1,
               is0, is1, os0, os1):
        wid = lax.axis_index("s") * nc + lax.axis_index("c")
        base = wid * b_per_w
        ibs, obs = [ib0, ib1], [ob0, ob1]
        isems, osems = [is0, is1], [os0, os1]

        pltpu.sync_copy(pos_hbm, pos_v)
        pltpu.async_copy(x_hbm.at[base], ibs[0], isems[0])
        pltpu.async_copy(x_hbm.at[base + 1], ibs[1], isems[1])

        for r in range(b_per_w):
            p = r % 2
            pltpu.make_async_copy(x_hbm.at[base + r], ibs[p], isems[p]).wait()
            if r >= 2:
                pltpu.make_async_copy(
                    obs[p], out_hbm.at[base + r - 2], osems[p]).wait()

            @plsc.parallel_loop(0, maxlen, unroll=2)
            def _add(i, _p=p):
                for j in range(d_chunks):
                    sl = pl.ds(j * _LANES, _LANES)
                    obs[_p][i, sl] = ibs[_p][i, sl] + pos_v[i, sl]

            pltpu.async_copy(obs[p], out_hbm.at[base + r], osems[p])
            if r + 2 < b_per_w:
                pltpu.async_copy(x_hbm.at[base + r + 2], ibs[p], isems[p])

        for r in (b_per_w - 2, b_per_w - 1):
            p = r % 2
            pltpu.make_async_copy(obs[p], out_hbm.at[base + r], osems[p]).wait()

    return sc_add


def kernel(x, pos_emb):
    batch, maxlen, dim = x.shape
    return _make_sc_add(batch, maxlen, dim)(x, pos_emb)
